# Initial kernel scaffold; baseline (speedup 1.0000x reference)
import jax
import jax.numpy as jnp
from jax.experimental import pallas as pl


def kernel(x, edge_index, W1, b1, W2, b2):
    raise NotImplementedError("write your pallas kernel here")


# speedup vs baseline: 12.6455x; 12.6455x over previous
import functools

import jax
import jax.numpy as jnp
from jax import lax
from jax.experimental import pallas as pl
from jax.experimental.pallas import tpu as pltpu
from jax.experimental.pallas import tpu_sc as plsc

N = 10000
D = 128
HD = D // 2
NC = 2
NS = 16
L = 16
NW = NC * NS
CH = 128
PT = 640
NPAD = NS * PT
DUMMY = N

_mesh = lambda: plsc.VectorSubcoreMesh(core_axis_name="c", subcore_axis_name="s")


def _sc_degree(dst_p, k):
    @functools.partial(
        pl.kernel,
        out_type=jax.ShapeDtypeStruct((NC, NPAD, L), jnp.float32),
        mesh=_mesh(),
        scratch_types=[
            pltpu.VMEM((k, CH), jnp.int32),
            pltpu.VMEM((CH, L), jnp.float32),
            pltpu.VMEM((CH, L), jnp.float32),
            pltpu.VMEM_SHARED((NPAD, L), jnp.float32),
        ],
    )
    def deg_kernel(dst_hbm, out_hbm, dst_v, ones_v, zeros_v, deg_sh):
        c = lax.axis_index("c")
        s = lax.axis_index("s")
        wid = c * NS + s
        pltpu.sync_copy(dst_hbm.at[wid], dst_v)

        @pl.loop(0, CH)
        def _(r):
            ones_v[r, :] = jnp.ones((L,), jnp.float32)
            zeros_v[r, :] = jnp.zeros((L,), jnp.float32)

        @pl.loop(0, PT, step=CH)
        def _(r0):
            pltpu.sync_copy(zeros_v, deg_sh.at[pl.ds(s * PT + r0, CH)])

        plsc.subcore_barrier()

        @pl.loop(0, k)
        def _(j):
            pltpu.sync_copy(ones_v, deg_sh.at[dst_v.at[j]], add=True)

        plsc.subcore_barrier()

        @pl.loop(0, PT, step=CH)
        def _(r0):
            pltpu.sync_copy(deg_sh.at[pl.ds(s * PT + r0, CH)],
                            out_hbm.at[c].at[pl.ds(s * PT + r0, CH)])

    return deg_kernel(dst_p)


def _sc_scatter(g_split, src_p, dst_p, k):
    @functools.partial(
        pl.kernel,
        out_type=jax.ShapeDtypeStruct((NC, NPAD, HD), jnp.float32),
        mesh=_mesh(),
        scratch_types=[
            pltpu.VMEM((k, CH), jnp.int32),
            pltpu.VMEM((k, CH), jnp.int32),
            pltpu.VMEM((CH, HD), jnp.float32),
            pltpu.VMEM((CH, HD), jnp.float32),
            pltpu.VMEM((CH, HD), jnp.float32),
            pltpu.VMEM((CH, HD), jnp.float32),
            pltpu.VMEM_SHARED((NPAD, HD), jnp.float32),
            pltpu.SemaphoreType.DMA,
            pltpu.SemaphoreType.DMA,
            pltpu.SemaphoreType.DMA,
            pltpu.SemaphoreType.DMA,
            pltpu.SemaphoreType.DMA,
            pltpu.SemaphoreType.DMA,
            pltpu.SemaphoreType.DMA,
            pltpu.SemaphoreType.DMA,
        ],
        compiler_params=pltpu.CompilerParams(use_tc_tiling_on_sc=False),
    )
    def scat_kernel(g_hbm, src_hbm, dst_hbm, out_hbm,
                    src_v, dst_v, buf0, buf1, buf2, buf3, acc_sh,
                    gsem0, gsem1, gsem2, gsem3,
                    ssem0, ssem1, ssem2, ssem3):
        c = lax.axis_index("c")
        s = lax.axis_index("s")
        pltpu.sync_copy(src_hbm.at[s], src_v)
        pltpu.sync_copy(dst_hbm.at[s], dst_v)

        @pl.loop(0, CH)
        def _(r):
            @pl.loop(0, HD, step=L)
            def _(l):
                buf0[r, pl.ds(l, L)] = jnp.zeros((L,), jnp.float32)

        @pl.loop(0, PT, step=CH)
        def _(r0):
            pltpu.sync_copy(buf0, acc_sh.at[pl.ds(s * PT + r0, CH)])

        plsc.subcore_barrier()

        tab = g_hbm.at[c]
        bufs = (buf0, buf1, buf2, buf3)
        gsems = (gsem0, gsem1, gsem2, gsem3)
        ssems = (ssem0, ssem1, ssem2, ssem3)
        NB = 4
        for b in range(NB):
            pltpu.async_copy(tab.at[src_v.at[b]], bufs[b], gsems[b])

        @pl.loop(0, k, step=NB)
        def _(j):
            for b in range(NB):
                pltpu.make_async_copy(tab.at[src_v.at[j + b]], bufs[b],
                                      gsems[b]).wait()
                pltpu.async_copy(bufs[b], acc_sh.at[dst_v.at[j + b]],
                                 ssems[b], add=True)
            for b in range(NB):
                pltpu.make_async_copy(bufs[b], acc_sh.at[dst_v.at[j + b]],
                                      ssems[b]).wait()

                @pl.when(j + NB + b < k)
                def _():
                    pltpu.async_copy(tab.at[src_v.at[j + NB + b]], bufs[b],
                                     gsems[b])

        plsc.subcore_barrier()

        @pl.loop(0, PT, step=CH)
        def _(r0):
            pltpu.sync_copy(acc_sh.at[pl.ds(s * PT + r0, CH)],
                            out_hbm.at[c].at[pl.ds(s * PT + r0, CH)])

    return scat_kernel(g_split, src_p, dst_p)


_ROWS = 1000


def _tc_matmul(x, W):
    def body(x_ref, w_ref, o_ref):
        o_ref[...] = jnp.dot(x_ref[...], w_ref[...],
                             preferred_element_type=jnp.float32)

    return pl.pallas_call(
        body,
        grid=(N // _ROWS,),
        in_specs=[pl.BlockSpec((_ROWS, D), lambda i: (i, 0)),
                  pl.BlockSpec((D, D), lambda i: (0, 0))],
        out_specs=pl.BlockSpec((_ROWS, D), lambda i: (i, 0)),
        out_shape=jax.ShapeDtypeStruct((N, D), jnp.float32),
    )(x, W)


def _tc_scale(xw, d0, d1):
    def body(xw_ref, d0_ref, d1_ref, g_ref, dv_ref):
        deg = d0_ref[...][:, 0:1] + d1_ref[...][:, 0:1] + 1.0
        dinv = lax.rsqrt(deg)
        g_ref[...] = xw_ref[...] * dinv
        dv_ref[...] = jnp.broadcast_to(dinv, dv_ref.shape)

    return pl.pallas_call(
        body,
        grid=(N // _ROWS,),
        in_specs=[pl.BlockSpec((_ROWS, D), lambda i: (i, 0)),
                  pl.BlockSpec((_ROWS, L), lambda i: (i, 0)),
                  pl.BlockSpec((_ROWS, L), lambda i: (i, 0))],
        out_specs=[pl.BlockSpec((_ROWS, D), lambda i: (i, 0)),
                   pl.BlockSpec((_ROWS, D), lambda i: (i, 0))],
        out_shape=[jax.ShapeDtypeStruct((N, D), jnp.float32),
                   jax.ShapeDtypeStruct((N, D), jnp.float32)],
    )(xw, d0, d1)


def _tc_mid(a_lo, a_hi, g1, dinv, b1, W2):
    def body(lo_ref, hi_ref, g_ref, dv_ref, b_ref, w_ref, o_ref):
        acc = jnp.concatenate([lo_ref[...], hi_ref[...]], axis=1)
        h = jnp.maximum(dv_ref[...] * (acc + g_ref[...]) + b_ref[...], 0.0)
        o_ref[...] = dv_ref[...] * jnp.dot(h, w_ref[...],
                                           preferred_element_type=jnp.float32)

    return pl.pallas_call(
        body,
        grid=(N // _ROWS,),
        in_specs=[pl.BlockSpec((_ROWS, HD), lambda i: (i, 0)),
                  pl.BlockSpec((_ROWS, HD), lambda i: (i, 0)),
                  pl.BlockSpec((_ROWS, D), lambda i: (i, 0)),
                  pl.BlockSpec((_ROWS, D), lambda i: (i, 0)),
                  pl.BlockSpec((1, D), lambda i: (0, 0)),
                  pl.BlockSpec((D, D), lambda i: (0, 0))],
        out_specs=pl.BlockSpec((_ROWS, D), lambda i: (i, 0)),
        out_shape=jax.ShapeDtypeStruct((N, D), jnp.float32),
    )(a_lo, a_hi, g1, dinv, b1, W2)


def _tc_final(a_lo, a_hi, g2, dinv, b2):
    def body(lo_ref, hi_ref, g_ref, dv_ref, b_ref, o_ref):
        acc = jnp.concatenate([lo_ref[...], hi_ref[...]], axis=1)
        o_ref[...] = dv_ref[...] * (acc + g_ref[...]) + b_ref[...]

    return pl.pallas_call(
        body,
        grid=(N // _ROWS,),
        in_specs=[pl.BlockSpec((_ROWS, HD), lambda i: (i, 0)),
                  pl.BlockSpec((_ROWS, HD), lambda i: (i, 0)),
                  pl.BlockSpec((_ROWS, D), lambda i: (i, 0)),
                  pl.BlockSpec((_ROWS, D), lambda i: (i, 0)),
                  pl.BlockSpec((1, D), lambda i: (0, 0))],
        out_specs=pl.BlockSpec((_ROWS, D), lambda i: (i, 0)),
        out_shape=jax.ShapeDtypeStruct((N, D), jnp.float32),
    )(a_lo, a_hi, g2, dinv, b2)


def _split(g):
    return jnp.stack([g[:, :HD], g[:, HD:]])


def kernel(x, edge_index, W1, b1, W2, b2):
    ei = edge_index.astype(jnp.int32)
    src, dst = ei[0], ei[1]
    e = src.shape[0]
    kd = -(-e // (NW * CH))
    kd += kd % 2
    ks = 2 * kd
    pad = NW * kd * CH - e
    src_f = jnp.concatenate([src, jnp.zeros((pad,), jnp.int32)])
    dst_f = jnp.concatenate([dst, jnp.full((pad,), DUMMY, jnp.int32)])
    src_p = src_f.reshape(NS, ks, CH)
    dst_p = dst_f.reshape(NS, ks, CH)

    degp = _sc_degree(dst_f.reshape(NW, kd, CH), kd)
    xw1 = _tc_matmul(x, W1)
    g1, dinv = _tc_scale(xw1, degp[0, :N], degp[1, :N])
    acc1 = _sc_scatter(_split(g1), src_p, dst_p, ks)
    g2 = _tc_mid(acc1[0, :N], acc1[1, :N], g1, dinv, b1.reshape(1, D), W2)
    acc2 = _sc_scatter(_split(g2), src_p, dst_p, ks)
    return _tc_final(acc2[0, :N], acc2[1, :N], g2, dinv, b2.reshape(1, D))


# speedup vs baseline: 13.3085x; 1.0524x over previous
import functools

import jax
import jax.numpy as jnp
from jax import lax
from jax.experimental import pallas as pl
from jax.experimental.pallas import tpu as pltpu
from jax.experimental.pallas import tpu_sc as plsc

N = 10000
D = 128
HD = D // 2
NC = 2
NS = 16
L = 16
NW = NC * NS
CH = 128
PT = 640
NPAD = NS * PT
DUMMY = N
_NB = 5

_mesh = lambda: plsc.VectorSubcoreMesh(core_axis_name="c", subcore_axis_name="s")


def _sc_degree(dst_p, k):
    @functools.partial(
        pl.kernel,
        out_type=jax.ShapeDtypeStruct((NC, NPAD, L), jnp.float32),
        mesh=_mesh(),
        scratch_types=[
            pltpu.VMEM((k, CH), jnp.int32),
            pltpu.VMEM((CH, L), jnp.float32),
            pltpu.VMEM((CH, L), jnp.float32),
            pltpu.VMEM_SHARED((NPAD, L), jnp.float32),
        ],
    )
    def deg_kernel(dst_hbm, out_hbm, dst_v, ones_v, zeros_v, deg_sh):
        c = lax.axis_index("c")
        s = lax.axis_index("s")
        wid = c * NS + s
        pltpu.sync_copy(dst_hbm.at[wid], dst_v)

        @pl.loop(0, CH)
        def _(r):
            ones_v[r, :] = jnp.ones((L,), jnp.float32)
            zeros_v[r, :] = jnp.zeros((L,), jnp.float32)

        @pl.loop(0, PT, step=CH)
        def _(r0):
            pltpu.sync_copy(zeros_v, deg_sh.at[pl.ds(s * PT + r0, CH)])

        plsc.subcore_barrier()

        @pl.loop(0, k)
        def _(j):
            pltpu.sync_copy(ones_v, deg_sh.at[dst_v.at[j]], add=True)

        plsc.subcore_barrier()

        @pl.loop(0, PT, step=CH)
        def _(r0):
            pltpu.sync_copy(deg_sh.at[pl.ds(s * PT + r0, CH)],
                            out_hbm.at[c].at[pl.ds(s * PT + r0, CH)])

    return deg_kernel(dst_p)


def _sc_scatter(g_split, src_p, dst_p, k):
    @functools.partial(
        pl.kernel,
        out_type=jax.ShapeDtypeStruct((NC, NPAD, HD), jnp.float32),
        mesh=_mesh(),
        scratch_types=[
            pltpu.VMEM((k, CH), jnp.int32),
            pltpu.VMEM((k, CH), jnp.int32),
        ] + [pltpu.VMEM((CH, HD), jnp.float32)] * _NB
          + [pltpu.VMEM_SHARED((NPAD, HD), jnp.float32)]
          + [pltpu.SemaphoreType.DMA] * (2 * _NB),
        compiler_params=pltpu.CompilerParams(use_tc_tiling_on_sc=False),
    )
    def scat_kernel(g_hbm, src_hbm, dst_hbm, out_hbm,
                    src_v, dst_v, *rest):
        bufs = rest[:_NB]
        acc_sh = rest[_NB]
        gsems = rest[_NB + 1:2 * _NB + 1]
        ssems = rest[2 * _NB + 1:]
        buf0 = bufs[0]
        c = lax.axis_index("c")
        s = lax.axis_index("s")
        pltpu.sync_copy(src_hbm.at[s], src_v)
        pltpu.sync_copy(dst_hbm.at[s], dst_v)

        @pl.loop(0, CH)
        def _(r):
            @pl.loop(0, HD, step=L)
            def _(l):
                buf0[r, pl.ds(l, L)] = jnp.zeros((L,), jnp.float32)

        @pl.loop(0, PT, step=CH)
        def _(r0):
            pltpu.sync_copy(buf0, acc_sh.at[pl.ds(s * PT + r0, CH)])

        plsc.subcore_barrier()

        tab = g_hbm.at[c]
        for b in range(_NB):
            pltpu.async_copy(tab.at[src_v.at[b]], bufs[b], gsems[b])

        @pl.loop(0, k, step=_NB)
        def _(j):
            for b in range(_NB):
                pltpu.make_async_copy(tab.at[src_v.at[j + b]], bufs[b],
                                      gsems[b]).wait()
                pltpu.async_copy(bufs[b], acc_sh.at[dst_v.at[j + b]],
                                 ssems[b], add=True)
            for b in range(_NB):
                pltpu.make_async_copy(bufs[b], acc_sh.at[dst_v.at[j + b]],
                                      ssems[b]).wait()

                @pl.when(j + _NB + b < k)
                def _():
                    pltpu.async_copy(tab.at[src_v.at[j + _NB + b]], bufs[b],
                                     gsems[b])

        plsc.subcore_barrier()

        @pl.loop(0, PT, step=CH)
        def _(r0):
            pltpu.sync_copy(acc_sh.at[pl.ds(s * PT + r0, CH)],
                            out_hbm.at[c].at[pl.ds(s * PT + r0, CH)])

    return scat_kernel(g_split, src_p, dst_p)


_ROWS = 1000


def _tc_matmul(x, W):
    def body(x_ref, w_ref, o_ref):
        o_ref[...] = jnp.dot(x_ref[...], w_ref[...],
                             preferred_element_type=jnp.float32)

    return pl.pallas_call(
        body,
        grid=(N // _ROWS,),
        in_specs=[pl.BlockSpec((_ROWS, D), lambda i: (i, 0)),
                  pl.BlockSpec((D, D), lambda i: (0, 0))],
        out_specs=pl.BlockSpec((_ROWS, D), lambda i: (i, 0)),
        out_shape=jax.ShapeDtypeStruct((N, D), jnp.float32),
    )(x, W)


def _tc_scale(xw, d0, d1):
    def body(xw_ref, d0_ref, d1_ref, g_ref, dv_ref):
        deg = d0_ref[...][:, 0:1] + d1_ref[...][:, 0:1] + 1.0
        dinv = lax.rsqrt(deg)
        g = xw_ref[...] * dinv
        g_ref[0] = g[:, :HD]
        g_ref[1] = g[:, HD:]
        dv_ref[...] = jnp.broadcast_to(dinv, dv_ref.shape)

    return pl.pallas_call(
        body,
        grid=(N // _ROWS,),
        in_specs=[pl.BlockSpec((_ROWS, D), lambda i: (i, 0)),
                  pl.BlockSpec((_ROWS, L), lambda i: (i, 0)),
                  pl.BlockSpec((_ROWS, L), lambda i: (i, 0))],
        out_specs=[pl.BlockSpec((NC, _ROWS, HD), lambda i: (0, i, 0)),
                   pl.BlockSpec((_ROWS, D), lambda i: (i, 0))],
        out_shape=[jax.ShapeDtypeStruct((NC, N, HD), jnp.float32),
                   jax.ShapeDtypeStruct((N, D), jnp.float32)],
    )(xw, d0, d1)


def _tc_mid(acc, gs, dinv, b1, W2):
    def body(a_ref, g_ref, dv_ref, b_ref, w_ref, o_ref):
        s = jnp.concatenate([a_ref[0] + g_ref[0], a_ref[1] + g_ref[1]],
                            axis=1)
        h = jnp.maximum(dv_ref[...] * s + b_ref[...], 0.0)
        g2 = dv_ref[...] * jnp.dot(h, w_ref[...],
                                   preferred_element_type=jnp.float32)
        o_ref[0] = g2[:, :HD]
        o_ref[1] = g2[:, HD:]

    return pl.pallas_call(
        body,
        grid=(N // _ROWS,),
        in_specs=[pl.BlockSpec((NC, _ROWS, HD), lambda i: (0, i, 0)),
                  pl.BlockSpec((NC, _ROWS, HD), lambda i: (0, i, 0)),
                  pl.BlockSpec((_ROWS, D), lambda i: (i, 0)),
                  pl.BlockSpec((1, D), lambda i: (0, 0)),
                  pl.BlockSpec((D, D), lambda i: (0, 0))],
        out_specs=pl.BlockSpec((NC, _ROWS, HD), lambda i: (0, i, 0)),
        out_shape=jax.ShapeDtypeStruct((NC, N, HD), jnp.float32),
    )(acc, gs, dinv, b1, W2)


def _tc_final(acc, gs, dinv, b2):
    def body(a_ref, g_ref, dv_ref, b_ref, o_ref):
        s = jnp.concatenate([a_ref[0] + g_ref[0], a_ref[1] + g_ref[1]],
                            axis=1)
        o_ref[...] = dv_ref[...] * s + b_ref[...]

    return pl.pallas_call(
        body,
        grid=(N // _ROWS,),
        in_specs=[pl.BlockSpec((NC, _ROWS, HD), lambda i: (0, i, 0)),
                  pl.BlockSpec((NC, _ROWS, HD), lambda i: (0, i, 0)),
                  pl.BlockSpec((_ROWS, D), lambda i: (i, 0)),
                  pl.BlockSpec((1, D), lambda i: (0, 0))],
        out_specs=pl.BlockSpec((_ROWS, D), lambda i: (i, 0)),
        out_shape=jax.ShapeDtypeStruct((N, D), jnp.float32),
    )(acc, gs, dinv, b2)


def kernel(x, edge_index, W1, b1, W2, b2):
    ei = edge_index.astype(jnp.int32)
    src, dst = ei[0], ei[1]
    e = src.shape[0]
    kd = -(-e // (NW * CH))
    while (2 * kd) % _NB:
        kd += 1
    ks = 2 * kd
    pad = NW * kd * CH - e
    src_f = jnp.concatenate([src, jnp.zeros((pad,), jnp.int32)])
    dst_f = jnp.concatenate([dst, jnp.full((pad,), DUMMY, jnp.int32)])
    src_p = src_f.reshape(NS, ks, CH)
    dst_p = dst_f.reshape(NS, ks, CH)

    degp = _sc_degree(dst_f.reshape(NW, kd, CH), kd)
    xw1 = _tc_matmul(x, W1)
    g1s, dinv = _tc_scale(xw1, degp[0, :N], degp[1, :N])
    acc1 = _sc_scatter(g1s, src_p, dst_p, ks)
    g2s = _tc_mid(acc1, g1s, dinv, b1.reshape(1, D), W2)
    acc2 = _sc_scatter(g2s, src_p, dst_p, ks)
    return _tc_final(acc2, g2s, dinv, b2.reshape(1, D))


# speedup vs baseline: 13.6482x; 1.0255x over previous
import functools

import jax
import jax.numpy as jnp
from jax import lax
from jax.experimental import pallas as pl
from jax.experimental.pallas import tpu as pltpu
from jax.experimental.pallas import tpu_sc as plsc

N = 10000
D = 128
HD = D // 2
NC = 2
NS = 16
L = 16
NW = NC * NS
CH = 128
PT = 640
NPAD = NS * PT
DUMMY = N
_NB = 5

_mesh = lambda: plsc.VectorSubcoreMesh(core_axis_name="c", subcore_axis_name="s")


def _sc_degree(dst_p, k):
    @functools.partial(
        pl.kernel,
        out_type=jax.ShapeDtypeStruct((NC, NPAD, L), jnp.float32),
        mesh=_mesh(),
        scratch_types=[
            pltpu.VMEM((k, CH), jnp.int32),
            pltpu.VMEM((CH, L), jnp.float32),
            pltpu.VMEM((CH, L), jnp.float32),
            pltpu.VMEM_SHARED((NPAD, L), jnp.float32),
        ],
    )
    def deg_kernel(dst_hbm, out_hbm, dst_v, ones_v, zeros_v, deg_sh):
        c = lax.axis_index("c")
        s = lax.axis_index("s")
        wid = c * NS + s
        pltpu.sync_copy(dst_hbm.at[wid], dst_v)

        @pl.loop(0, CH)
        def _(r):
            ones_v[r, :] = jnp.ones((L,), jnp.float32)
            zeros_v[r, :] = jnp.zeros((L,), jnp.float32)

        @pl.loop(0, PT, step=CH)
        def _(r0):
            pltpu.sync_copy(zeros_v, deg_sh.at[pl.ds(s * PT + r0, CH)])

        plsc.subcore_barrier()

        @pl.loop(0, k)
        def _(j):
            pltpu.sync_copy(ones_v, deg_sh.at[dst_v.at[j]], add=True)

        plsc.subcore_barrier()

        @pl.loop(0, PT, step=CH)
        def _(r0):
            pltpu.sync_copy(deg_sh.at[pl.ds(s * PT + r0, CH)],
                            out_hbm.at[c].at[pl.ds(s * PT + r0, CH)])

    return deg_kernel(dst_p)


def _sc_scatter(g_split, src_p, dst_p, k):
    @functools.partial(
        pl.kernel,
        out_type=jax.ShapeDtypeStruct((NC, NPAD, HD), jnp.float32),
        mesh=_mesh(),
        scratch_types=[
            pltpu.VMEM((k, CH), jnp.int32),
            pltpu.VMEM((k, CH), jnp.int32),
        ] + [pltpu.VMEM((CH, HD), jnp.float32)] * _NB
          + [pltpu.VMEM_SHARED((NPAD, HD), jnp.float32)]
          + [pltpu.SemaphoreType.DMA] * (2 * _NB),
        compiler_params=pltpu.CompilerParams(use_tc_tiling_on_sc=False),
    )
    def scat_kernel(g_hbm, src_hbm, dst_hbm, out_hbm,
                    src_v, dst_v, *rest):
        bufs = rest[:_NB]
        acc_sh = rest[_NB]
        gsems = rest[_NB + 1:2 * _NB + 1]
        ssems = rest[2 * _NB + 1:]
        buf0 = bufs[0]
        c = lax.axis_index("c")
        s = lax.axis_index("s")
        pltpu.async_copy(src_hbm.at[s], src_v, gsems[0])
        pltpu.async_copy(dst_hbm.at[s], dst_v, gsems[1])

        @pl.loop(0, CH)
        def _(r):
            @pl.loop(0, HD, step=L)
            def _(l):
                buf0[r, pl.ds(l, L)] = jnp.zeros((L,), jnp.float32)

        @pl.loop(0, PT, step=CH)
        def _(r0):
            pltpu.sync_copy(buf0, acc_sh.at[pl.ds(s * PT + r0, CH)])

        pltpu.make_async_copy(src_hbm.at[s], src_v, gsems[0]).wait()
        pltpu.make_async_copy(dst_hbm.at[s], dst_v, gsems[1]).wait()
        plsc.subcore_barrier()

        tab = g_hbm.at[c]
        for b in range(_NB):
            pltpu.async_copy(tab.at[src_v.at[b]], bufs[b], gsems[b])

        @pl.loop(0, k, step=_NB)
        def _(j):
            for b in range(_NB):
                pltpu.make_async_copy(tab.at[src_v.at[j + b]], bufs[b],
                                      gsems[b]).wait()
                pltpu.async_copy(bufs[b], acc_sh.at[dst_v.at[j + b]],
                                 ssems[b], add=True)
            for b in range(_NB):
                pltpu.make_async_copy(bufs[b], acc_sh.at[dst_v.at[j + b]],
                                      ssems[b]).wait()

                @pl.when(j + _NB + b < k)
                def _():
                    pltpu.async_copy(tab.at[src_v.at[j + _NB + b]], bufs[b],
                                     gsems[b])

        plsc.subcore_barrier()

        @pl.loop(0, PT, step=CH)
        def _(r0):
            pltpu.sync_copy(acc_sh.at[pl.ds(s * PT + r0, CH)],
                            out_hbm.at[c].at[pl.ds(s * PT + r0, CH)])

    return scat_kernel(g_split, src_p, dst_p)


_ROWS = 1000


def _tc_front(x, W, d0, d1):
    def body(x_ref, w_ref, d0_ref, d1_ref, g_ref, dv_ref):
        deg = d0_ref[...][:, 0:1] + d1_ref[...][:, 0:1] + 1.0
        dinv = lax.rsqrt(deg)
        g = jnp.dot(x_ref[...], w_ref[...],
                    preferred_element_type=jnp.float32) * dinv
        g_ref[0] = g[:, :HD]
        g_ref[1] = g[:, HD:]
        dv_ref[...] = jnp.broadcast_to(dinv, dv_ref.shape)

    return pl.pallas_call(
        body,
        grid=(N // _ROWS,),
        in_specs=[pl.BlockSpec((_ROWS, D), lambda i: (i, 0)),
                  pl.BlockSpec((D, D), lambda i: (0, 0)),
                  pl.BlockSpec((_ROWS, L), lambda i: (i, 0)),
                  pl.BlockSpec((_ROWS, L), lambda i: (i, 0))],
        out_specs=[pl.BlockSpec((NC, _ROWS, HD), lambda i: (0, i, 0)),
                   pl.BlockSpec((_ROWS, D), lambda i: (i, 0))],
        out_shape=[jax.ShapeDtypeStruct((NC, N, HD), jnp.float32),
                   jax.ShapeDtypeStruct((N, D), jnp.float32)],
    )(x, W, d0, d1)


def _tc_mid(acc, gs, dinv, b1, W2):
    def body(a_ref, g_ref, dv_ref, b_ref, w_ref, o_ref):
        s = jnp.concatenate([a_ref[0] + g_ref[0], a_ref[1] + g_ref[1]],
                            axis=1)
        h = jnp.maximum(dv_ref[...] * s + b_ref[...], 0.0)
        g2 = dv_ref[...] * jnp.dot(h, w_ref[...],
                                   preferred_element_type=jnp.float32)
        o_ref[0] = g2[:, :HD]
        o_ref[1] = g2[:, HD:]

    return pl.pallas_call(
        body,
        grid=(N // _ROWS,),
        in_specs=[pl.BlockSpec((NC, _ROWS, HD), lambda i: (0, i, 0)),
                  pl.BlockSpec((NC, _ROWS, HD), lambda i: (0, i, 0)),
                  pl.BlockSpec((_ROWS, D), lambda i: (i, 0)),
                  pl.BlockSpec((1, D), lambda i: (0, 0)),
                  pl.BlockSpec((D, D), lambda i: (0, 0))],
        out_specs=pl.BlockSpec((NC, _ROWS, HD), lambda i: (0, i, 0)),
        out_shape=jax.ShapeDtypeStruct((NC, N, HD), jnp.float32),
    )(acc, gs, dinv, b1, W2)


def _tc_final(acc, gs, dinv, b2):
    def body(a_ref, g_ref, dv_ref, b_ref, o_ref):
        s = jnp.concatenate([a_ref[0] + g_ref[0], a_ref[1] + g_ref[1]],
                            axis=1)
        o_ref[...] = dv_ref[...] * s + b_ref[...]

    return pl.pallas_call(
        body,
        grid=(N // _ROWS,),
        in_specs=[pl.BlockSpec((NC, _ROWS, HD), lambda i: (0, i, 0)),
                  pl.BlockSpec((NC, _ROWS, HD), lambda i: (0, i, 0)),
                  pl.BlockSpec((_ROWS, D), lambda i: (i, 0)),
                  pl.BlockSpec((1, D), lambda i: (0, 0))],
        out_specs=pl.BlockSpec((_ROWS, D), lambda i: (i, 0)),
        out_shape=jax.ShapeDtypeStruct((N, D), jnp.float32),
    )(acc, gs, dinv, b2)


def kernel(x, edge_index, W1, b1, W2, b2):
    ei = edge_index.astype(jnp.int32)
    src, dst = ei[0], ei[1]
    e = src.shape[0]
    kd = -(-e // (NW * CH))
    while (2 * kd) % _NB:
        kd += 1
    ks = 2 * kd
    pad = NW * kd * CH - e
    src_f = jnp.concatenate([src, jnp.zeros((pad,), jnp.int32)])
    dst_f = jnp.concatenate([dst, jnp.full((pad,), DUMMY, jnp.int32)])
    src_p = src_f.reshape(NS, ks, CH)
    dst_p = dst_f.reshape(NS, ks, CH)

    degp = _sc_degree(dst_f.reshape(NW, kd, CH), kd)
    g1s, dinv = _tc_front(x, W1, degp[0, :N], degp[1, :N])
    acc1 = _sc_scatter(g1s, src_p, dst_p, ks)
    g2s = _tc_mid(acc1, g1s, dinv, b1.reshape(1, D), W2)
    acc2 = _sc_scatter(g2s, src_p, dst_p, ks)
    return _tc_final(acc2, g2s, dinv, b2.reshape(1, D))
